# Initial kernel scaffold; baseline (speedup 1.0000x reference)
import jax
import jax.numpy as jnp
from jax.experimental import pallas as pl


def kernel(cls_head_0, cls_head_1, cls_head_2, cls_head_3, cls_head_4, reg_head_0, reg_head_1, reg_head_2, reg_head_3, reg_head_4, center_head_0, center_head_1, center_head_2, center_head_3, center_head_4):
    raise NotImplementedError("write your pallas kernel here")


# speedup vs baseline: 3.5506x; 3.5506x over previous
import functools

import jax
import jax.numpy as jnp
from jax.experimental import pallas as pl
from jax.experimental.pallas import tpu as pltpu

_IMAGE_W = 1024
_IMAGE_H = 1024
_STRIDES = (8, 16, 32, 64, 128)
_TOP_N = 1000
_MIN_SCORE = 0.05
_NMS_THR = 0.6
_MAX_DET = 100
_B = 4
_C = 80

_HW = tuple((_IMAGE_H // s) * (_IMAGE_W // s) for s in _STRIDES)
_TOTAL = sum(_HW)
_NPAD = ((_TOTAL + 127) // 128) * 128
_OFFS = (0, 16384, 20480, 21504, 21760)


def _decode_body(w_mask, w_shift, stride, ch, cls_ref, r0_ref, r1_ref,
                 r2_ref, r3_ref, ctr_ref,
                 s_ref, c_ref, x1_ref, y1_ref, x2_ref, y2_ref):
    i = pl.program_id(0)
    cls = jax.nn.sigmoid(cls_ref[...])
    m = jnp.max(cls, axis=2)
    lane_c = jax.lax.broadcasted_iota(jnp.int32, cls.shape, 2)
    cls_idx = jnp.min(jnp.where(cls == m[:, :, None], lane_c, _C), axis=2)
    ctr = jax.nn.sigmoid(ctr_ref[...])
    s_ref[...] = jnp.sqrt(m * ctr)
    c_ref[...] = cls_idx.astype(jnp.float32)

    g = i * ch + jax.lax.broadcasted_iota(jnp.int32, (_B, ch), 1)
    px = ((g & w_mask).astype(jnp.float32) + 0.5) * stride
    py = ((g >> w_shift).astype(jnp.float32) + 0.5) * stride
    x1_ref[...] = jnp.maximum(px - jnp.exp(r0_ref[...]), 0.0)
    y1_ref[...] = jnp.maximum(py - jnp.exp(r1_ref[...]), 0.0)
    x2_ref[...] = jnp.minimum(px + jnp.exp(r2_ref[...]), _IMAGE_W - 1.0)
    y2_ref[...] = jnp.minimum(py + jnp.exp(r3_ref[...]), _IMAGE_H - 1.0)


def _decode_level(cls_h, reg_h, ctr_h, stride):
    b, h, w, c = cls_h.shape
    hw = h * w
    ch = min(hw, 2048)
    grid = hw // ch
    w_shift = w.bit_length() - 1
    body = functools.partial(_decode_body, w - 1, w_shift, float(stride), ch)
    out2d = jax.ShapeDtypeStruct((b, hw), jnp.float32)
    spec2d = pl.BlockSpec((b, ch), lambda i: (0, i))
    reg = reg_h.reshape(b, hw, 4)
    return pl.pallas_call(
        body,
        grid=(grid,),
        in_specs=[pl.BlockSpec((b, ch, c), lambda i: (0, i, 0))] + [spec2d] * 5,
        out_specs=[spec2d] * 6,
        out_shape=[out2d] * 6,
    )(cls_h.reshape(b, hw, c), reg[..., 0], reg[..., 1], reg[..., 2],
      reg[..., 3], ctr_h.reshape(b, hw))


def _nms_body(s_ref, c_ref, x1_ref, y1_ref, x2_ref, y2_ref,
              ks_ref, kc_ref, kx1_ref, ky1_ref, kx2_ref, ky2_ref, sm_ref):
    scores = s_ref[...]
    lane = jax.lax.broadcasted_iota(jnp.int32, (_B, _NPAD), 1)

    bits = jax.lax.bitcast_convert_type(scores, jnp.int32)
    threshs = []
    for lvl in range(3):
        seg = bits[:, _OFFS[lvl]:_OFFS[lvl] + _HW[lvl]]

        def bs_step(_, carry, seg=seg):
            lo, hi = carry
            mid = (lo + hi) >> 1
            cnt = jnp.sum((seg >= mid).astype(jnp.int32), axis=1,
                          keepdims=True)
            take = cnt >= _TOP_N
            return jnp.where(take, mid, lo), jnp.where(take, hi, mid)

        lo0 = jnp.zeros((_B, 1), jnp.int32)
        hi0 = jnp.full((_B, 1), 0x40000000, jnp.int32)
        lo, _ = jax.lax.fori_loop(0, 31, bs_step, (lo0, hi0))
        threshs.append(lo)

    tmap = jnp.where(
        lane < _OFFS[1], threshs[0],
        jnp.where(lane < _OFFS[2], threshs[1],
                  jnp.where(lane < _OFFS[3], threshs[2], 0)))
    sm_ref[...] = jnp.where((scores > _MIN_SCORE) & (bits >= tmap),
                            scores, -1.0)

    x1 = x1_ref[...]
    y1 = y1_ref[...]
    x2 = x2_ref[...]
    y2 = y2_ref[...]
    cls_f = c_ref[...]
    areas = (x2 - x1) * (y2 - y1)
    lane100 = jax.lax.broadcasted_iota(jnp.int32, (_B, _MAX_DET), 1)
    zeros100 = jnp.zeros((_B, _MAX_DET), jnp.float32)

    def step(i, accs):
        a_s, a_c, a_x1, a_y1, a_x2, a_y2 = accs
        sm = sm_ref[...]
        m = jnp.max(sm, axis=1, keepdims=True)
        idx = jnp.min(jnp.where(sm == m, lane, _NPAD), axis=1, keepdims=True)
        onehot = lane == idx
        ohf = onehot.astype(jnp.float32)
        selx1 = jnp.sum(x1 * ohf, axis=1, keepdims=True)
        sely1 = jnp.sum(y1 * ohf, axis=1, keepdims=True)
        selx2 = jnp.sum(x2 * ohf, axis=1, keepdims=True)
        sely2 = jnp.sum(y2 * ohf, axis=1, keepdims=True)
        selc = jnp.sum(cls_f * ohf, axis=1, keepdims=True)
        inter = (jnp.maximum(jnp.minimum(x2, selx2) - jnp.maximum(x1, selx1),
                             0.0) *
                 jnp.maximum(jnp.minimum(y2, sely2) - jnp.maximum(y1, sely1),
                             0.0))
        box_area = (selx2 - selx1) * (sely2 - sely1)
        union = jnp.maximum(box_area + areas - inter, 1e-4)
        iou = inter / union
        sm_ref[...] = jnp.where((iou >= _NMS_THR) | onehot, -1.0, sm)
        kept = m > 0.0
        slot = lane100 == i
        a_s += jnp.where(slot, jnp.where(kept, m, -1.0), zeros100)
        a_c += jnp.where(slot, jnp.where(kept, selc, -1.0), zeros100)
        a_x1 += jnp.where(slot, jnp.where(kept, selx1, -1.0), zeros100)
        a_y1 += jnp.where(slot, jnp.where(kept, sely1, -1.0), zeros100)
        a_x2 += jnp.where(slot, jnp.where(kept, selx2, -1.0), zeros100)
        a_y2 += jnp.where(slot, jnp.where(kept, sely2, -1.0), zeros100)
        return a_s, a_c, a_x1, a_y1, a_x2, a_y2

    init = (zeros100,) * 6
    a_s, a_c, a_x1, a_y1, a_x2, a_y2 = jax.lax.fori_loop(
        0, _MAX_DET, step, init)
    ks_ref[...] = a_s
    kc_ref[...] = a_c
    kx1_ref[...] = a_x1
    ky1_ref[...] = a_y1
    kx2_ref[...] = a_x2
    ky2_ref[...] = a_y2


def _nms(scores, cls_f, x1, y1, x2, y2):
    out = jax.ShapeDtypeStruct((_B, _MAX_DET), jnp.float32)
    return pl.pallas_call(
        _nms_body,
        out_shape=[out] * 6,
        scratch_shapes=[pltpu.VMEM((_B, _NPAD), jnp.float32)],
    )(scores, cls_f, x1, y1, x2, y2)


def kernel(cls_head_0, cls_head_1, cls_head_2, cls_head_3, cls_head_4,
           reg_head_0, reg_head_1, reg_head_2, reg_head_3, reg_head_4,
           center_head_0, center_head_1, center_head_2, center_head_3,
           center_head_4):
    cls_heads = [cls_head_0, cls_head_1, cls_head_2, cls_head_3, cls_head_4]
    reg_heads = [reg_head_0, reg_head_1, reg_head_2, reg_head_3, reg_head_4]
    ctr_heads = [center_head_0, center_head_1, center_head_2, center_head_3,
                 center_head_4]
    parts = [[] for _ in range(6)]
    for cls_h, reg_h, ctr_h, stride in zip(cls_heads, reg_heads, ctr_heads,
                                           _STRIDES):
        outs = _decode_level(cls_h, reg_h, ctr_h, stride)
        for lst, o in zip(parts, outs):
            lst.append(o)
    npad = _NPAD - _TOTAL
    cat = []
    for k, lst in enumerate(parts):
        fill = -1.0 if k == 0 else 0.0
        lst.append(jnp.full((_B, npad), fill, jnp.float32))
        cat.append(jnp.concatenate(lst, axis=1))
    ks, kc, kx1, ky1, kx2, ky2 = _nms(*cat)
    kb = jnp.stack([kx1, ky1, kx2, ky2], axis=-1)
    return ks, kc, kb


# speedup vs baseline: 4.6269x; 1.3031x over previous
import functools

import jax
import jax.numpy as jnp
from jax.experimental import pallas as pl
from jax.experimental.pallas import tpu as pltpu

_IMAGE_W = 1024
_IMAGE_H = 1024
_STRIDES = (8, 16, 32, 64, 128)
_TOP_N = 1000
_MIN_SCORE = 0.05
_NMS_THR = 0.6
_MAX_DET = 100
_B = 4
_C = 80

_HW = tuple((_IMAGE_H // s) * (_IMAGE_W // s) for s in _STRIDES)
_TOTAL = sum(_HW)
_NS = 2816
_NPAD = 8 * _NS
_OFFS = (0, 16384, 20480, 21504, 21760)


def _decode_body(w_mask, w_shift, stride, ch, cls_ref, r0_ref, r1_ref,
                 r2_ref, r3_ref, ctr_ref,
                 s_ref, c_ref, x1_ref, y1_ref, x2_ref, y2_ref):
    i = pl.program_id(0)
    cls = jax.nn.sigmoid(cls_ref[...])
    m = jnp.max(cls, axis=2)
    lane_c = jax.lax.broadcasted_iota(jnp.int32, cls.shape, 2)
    cls_idx = jnp.min(jnp.where(cls == m[:, :, None], lane_c, _C), axis=2)
    ctr = jax.nn.sigmoid(ctr_ref[...])
    s_ref[...] = jnp.sqrt(m * ctr)
    c_ref[...] = cls_idx.astype(jnp.float32)

    g = i * ch + jax.lax.broadcasted_iota(jnp.int32, (_B, ch), 1)
    px = ((g & w_mask).astype(jnp.float32) + 0.5) * stride
    py = ((g >> w_shift).astype(jnp.float32) + 0.5) * stride
    x1_ref[...] = jnp.maximum(px - jnp.exp(r0_ref[...]), 0.0)
    y1_ref[...] = jnp.maximum(py - jnp.exp(r1_ref[...]), 0.0)
    x2_ref[...] = jnp.minimum(px + jnp.exp(r2_ref[...]), _IMAGE_W - 1.0)
    y2_ref[...] = jnp.minimum(py + jnp.exp(r3_ref[...]), _IMAGE_H - 1.0)


def _decode_level(cls_h, reg_h, ctr_h, stride):
    b, h, w, c = cls_h.shape
    hw = h * w
    ch = min(hw, 2048)
    grid = hw // ch
    w_shift = w.bit_length() - 1
    body = functools.partial(_decode_body, w - 1, w_shift, float(stride), ch)
    out2d = jax.ShapeDtypeStruct((b, hw), jnp.float32)
    spec2d = pl.BlockSpec((b, ch), lambda i: (0, i))
    reg = reg_h.reshape(b, hw, 4)
    return pl.pallas_call(
        body,
        grid=(grid,),
        in_specs=[pl.BlockSpec((b, ch, c), lambda i: (0, i, 0))] + [spec2d] * 5,
        out_specs=[spec2d] * 6,
        out_shape=[out2d] * 6,
    )(cls_h.reshape(b, hw, c), reg[..., 0], reg[..., 1], reg[..., 2],
      reg[..., 3], ctr_h.reshape(b, hw))


def _nms_body(s_ref, c_ref, x1_ref, y1_ref, x2_ref, y2_ref,
              ks_ref, kc_ref, kx1_ref, ky1_ref, kx2_ref, ky2_ref, sm_ref):
    shape3 = (_B, 8, _NS)
    scores = s_ref[...]
    g = (jax.lax.broadcasted_iota(jnp.int32, shape3, 1) * _NS +
         jax.lax.broadcasted_iota(jnp.int32, shape3, 2))

    bits = jax.lax.bitcast_convert_type(scores, jnp.int32)
    threshs = []
    for lvl in range(3):
        in_lvl = (g >= _OFFS[lvl]) & (g < _OFFS[lvl] + _HW[lvl])

        def bs_step(_, carry, in_lvl=in_lvl):
            lo, hi = carry
            mid = (lo + hi) >> 1
            cnt = jnp.sum(((bits >= mid) & in_lvl).astype(jnp.int32),
                          axis=(1, 2), keepdims=True)
            take = cnt >= _TOP_N
            return jnp.where(take, mid, lo), jnp.where(take, hi, mid)

        lo0 = jnp.zeros((_B, 1, 1), jnp.int32)
        hi0 = jnp.full((_B, 1, 1), 0x40000000, jnp.int32)
        lo, _ = jax.lax.fori_loop(0, 31, bs_step, (lo0, hi0))
        threshs.append(lo)

    tmap = jnp.where(
        g < _OFFS[1], threshs[0],
        jnp.where(g < _OFFS[2], threshs[1],
                  jnp.where(g < _OFFS[3], threshs[2], 0)))
    sm_ref[...] = jnp.where((scores > _MIN_SCORE) & (bits >= tmap),
                            scores, -1.0)

    x1 = x1_ref[...]
    y1 = y1_ref[...]
    x2 = x2_ref[...]
    y2 = y2_ref[...]
    cls_f = c_ref[...]
    areas = (x2 - x1) * (y2 - y1)
    lane100 = jax.lax.broadcasted_iota(jnp.int32, (_B, _MAX_DET), 1)
    zeros100 = jnp.zeros((_B, _MAX_DET), jnp.float32)

    def step(i, accs):
        a_s, a_c, a_x1, a_y1, a_x2, a_y2 = accs
        sm = sm_ref[...]
        m = jnp.max(sm, axis=(1, 2), keepdims=True)
        idx = jnp.min(jnp.where(sm == m, g, _NPAD), axis=(1, 2),
                      keepdims=True)
        onehot = g == idx
        ohf = onehot.astype(jnp.float32)
        selx1 = jnp.sum(x1 * ohf, axis=(1, 2), keepdims=True)
        sely1 = jnp.sum(y1 * ohf, axis=(1, 2), keepdims=True)
        selx2 = jnp.sum(x2 * ohf, axis=(1, 2), keepdims=True)
        sely2 = jnp.sum(y2 * ohf, axis=(1, 2), keepdims=True)
        selc = jnp.sum(cls_f * ohf, axis=(1, 2), keepdims=True)
        inter = (jnp.maximum(jnp.minimum(x2, selx2) - jnp.maximum(x1, selx1),
                             0.0) *
                 jnp.maximum(jnp.minimum(y2, sely2) - jnp.maximum(y1, sely1),
                             0.0))
        box_area = (selx2 - selx1) * (sely2 - sely1)
        union = jnp.maximum(box_area + areas - inter, 1e-4)
        iou = inter / union
        sm_ref[...] = jnp.where((iou >= _NMS_THR) | onehot, -1.0, sm)
        kept = m[:, 0] > 0.0
        slot = lane100 == i
        a_s += jnp.where(slot, jnp.where(kept, m[:, 0], -1.0), zeros100)
        a_c += jnp.where(slot, jnp.where(kept, selc[:, 0], -1.0), zeros100)
        a_x1 += jnp.where(slot, jnp.where(kept, selx1[:, 0], -1.0), zeros100)
        a_y1 += jnp.where(slot, jnp.where(kept, sely1[:, 0], -1.0), zeros100)
        a_x2 += jnp.where(slot, jnp.where(kept, selx2[:, 0], -1.0), zeros100)
        a_y2 += jnp.where(slot, jnp.where(kept, sely2[:, 0], -1.0), zeros100)
        return a_s, a_c, a_x1, a_y1, a_x2, a_y2

    init = (zeros100,) * 6
    a_s, a_c, a_x1, a_y1, a_x2, a_y2 = jax.lax.fori_loop(
        0, _MAX_DET, step, init)
    ks_ref[...] = a_s
    kc_ref[...] = a_c
    kx1_ref[...] = a_x1
    ky1_ref[...] = a_y1
    kx2_ref[...] = a_x2
    ky2_ref[...] = a_y2


def _nms(scores, cls_f, x1, y1, x2, y2):
    out = jax.ShapeDtypeStruct((_B, _MAX_DET), jnp.float32)
    return pl.pallas_call(
        _nms_body,
        out_shape=[out] * 6,
        scratch_shapes=[pltpu.VMEM((_B, 8, _NS), jnp.float32)],
    )(scores, cls_f, x1, y1, x2, y2)


def kernel(cls_head_0, cls_head_1, cls_head_2, cls_head_3, cls_head_4,
           reg_head_0, reg_head_1, reg_head_2, reg_head_3, reg_head_4,
           center_head_0, center_head_1, center_head_2, center_head_3,
           center_head_4):
    cls_heads = [cls_head_0, cls_head_1, cls_head_2, cls_head_3, cls_head_4]
    reg_heads = [reg_head_0, reg_head_1, reg_head_2, reg_head_3, reg_head_4]
    ctr_heads = [center_head_0, center_head_1, center_head_2, center_head_3,
                 center_head_4]
    parts = [[] for _ in range(6)]
    for cls_h, reg_h, ctr_h, stride in zip(cls_heads, reg_heads, ctr_heads,
                                           _STRIDES):
        outs = _decode_level(cls_h, reg_h, ctr_h, stride)
        for lst, o in zip(parts, outs):
            lst.append(o)
    npad = _NPAD - _TOTAL
    cat = []
    for k, lst in enumerate(parts):
        fill = -1.0 if k == 0 else 0.0
        lst.append(jnp.full((_B, npad), fill, jnp.float32))
        cat.append(jnp.concatenate(lst, axis=1).reshape(_B, 8, _NS))
    ks, kc, kx1, ky1, kx2, ky2 = _nms(*cat)
    kb = jnp.stack([kx1, ky1, kx2, ky2], axis=-1)
    return ks, kc, kb
